# baseline (speedup 1.0000x reference)
import functools
import math
import jax
import jax.numpy as jnp
from jax import lax
from jax.experimental import pallas as pl
from jax.experimental.pallas import tpu as pltpu
from jax.experimental.pallas import tpu_sc as plsc

N0, N1, N2, D, G = 10000, 80000, 20000, 128, 64


_NC, _NS = 2, 16
_NW = _NC * _NS
_BATCH = 128
_CE = 1024
_NST = _CE + 2 * _BATCH
_R_CAP = 11520


def _round_up(x, m):
    return (x + m - 1) // m * m


@functools.lru_cache(maxsize=None)
def _sc_spmm(e_pad, n_src, n_dst):
    assert e_pad % (_NS * _CE * 2) == 0
    ew = e_pad // _NS
    nchunk = ew // _CE
    n_pass = math.ceil(n_dst / (2 * _R_CAP))
    r = _round_up(math.ceil(n_dst / (2 * n_pass)), 256)
    rpt = r // 16
    zrows = rpt + 1

    mesh = plsc.VectorSubcoreMesh(core_axis_name="c", subcore_axis_name="s")

    @functools.partial(
        pl.kernel,
        out_type=jax.ShapeDtypeStruct((n_dst, D), jnp.float32),
        mesh=mesh,
        scratch_types=[
            pltpu.VMEM_SHARED((r + 16, D), jnp.float32),
            pltpu.VMEM((_CE,), jnp.int32),
            pltpu.VMEM((_CE,), jnp.int32),
            pltpu.VMEM((_CE,), jnp.float32),
            pltpu.VMEM((_CE,), jnp.int32),
            pltpu.VMEM((_CE,), jnp.int32),
            pltpu.VMEM((_CE,), jnp.float32),
            pltpu.VMEM((_NST,), jnp.int32),
            pltpu.VMEM((_NST // _BATCH + 1, _BATCH), jnp.int32),
            pltpu.VMEM((_NST,), jnp.float32),
            pltpu.VMEM((_BATCH, D), jnp.float32),
            pltpu.VMEM((_BATCH, D), jnp.float32),
            pltpu.VMEM((16, D), jnp.float32),
            pltpu.SemaphoreType.DMA,
            pltpu.SemaphoreType.DMA,
            pltpu.SemaphoreType.DMA,
            pltpu.SemaphoreType.DMA,
            pltpu.SemaphoreType.DMA,
            pltpu.SemaphoreType.DMA,
        ],
        compiler_params=pltpu.CompilerParams(needs_layout_passes=False),
    )
    def spmm_kernel(row_h, col_h, val_h, x_h, out_h, acc,
                    erA, ecA, evA, erB, ecB, evB,
                    stc, stl, stv, rows0, rows1, zbuf,
                    gsem0, gsem1, ssem0, ssem1, esemA, esemB):
        cid = lax.axis_index("c")
        sid = lax.axis_index("s")
        base_e = sid * ew
        iota = lax.broadcasted_iota(jnp.int32, (16,), 0)
        ones16 = jnp.ones((16,), jnp.int32)
        splat_idx = [jnp.full((16, 1), kk, jnp.int32) for kk in range(16)]
        dnums = lax.GatherDimensionNumbers(
            offset_dims=(), collapsed_slice_dims=(0,), start_index_map=(0,))

        def zero_z(i, _):
            for d in range(D // 16):
                zbuf[i, pl.ds(d * 16, 16)] = jnp.zeros((16,), jnp.float32)
            return 0

        lax.fori_loop(0, 16, zero_z, 0)

        def gissue(bi, rbuf, sem):
            pltpu.async_copy(x_h.at[stc.at[pl.ds(bi * _BATCH, _BATCH)]],
                             rbuf, sem)

        def gwait(bi, rbuf, sem):
            pltpu.make_async_copy(x_h.at[stc.at[pl.ds(bi * _BATCH, _BATCH)]],
                                  rbuf, sem).wait()

        def scale(bi, rbuf):
            boff = bi * _BATCH

            def body(k, _):
                vv = stv[pl.ds(boff + k * 16, 16)]
                for kk in range(16):
                    v = lax.gather(
                        vv, splat_idx[kk], dnums, (1,),
                        mode=lax.GatherScatterMode.PROMISE_IN_BOUNDS)
                    j = k * 16 + kk
                    for d in range(D // 16):
                        rbuf[j, pl.ds(d * 16, 16)] = (
                            rbuf[j, pl.ds(d * 16, 16)] * v)
                return 0

            lax.fori_loop(0, _BATCH // 16, body, 0)

        def swait(rbuf, sem):
            pltpu.make_async_copy(rbuf, acc.at[stl.at[0]], sem).wait()

        def run_batches(nb):
            @pl.when(nb > 0)
            def _():
                gissue(0, rows0, gsem0)

            @pl.when(nb > 1)
            def _():
                gissue(1, rows1, gsem1)

            def pair(k, _):
                b0 = 2 * k

                @pl.when(b0 < nb)
                def _():
                    gwait(b0, rows0, gsem0)
                    scale(b0, rows0)
                    pltpu.async_copy(rows0, acc.at[stl.at[b0]], ssem0,
                                     add=True)

                @pl.when(b0 + 2 < nb)
                def _():
                    swait(rows0, ssem0)
                    gissue(b0 + 2, rows0, gsem0)

                @pl.when(b0 + 1 < nb)
                def _():
                    gwait(b0 + 1, rows1, gsem1)
                    scale(b0 + 1, rows1)
                    pltpu.async_copy(rows1, acc.at[stl.at[b0 + 1]], ssem1,
                                     add=True)

                @pl.when(b0 + 3 < nb)
                def _():
                    swait(rows1, ssem1)
                    gissue(b0 + 3, rows1, gsem1)

                return 0

            lax.fori_loop(0, (nb + 1) // 2, pair, 0)

            @pl.when(nb >= 1)
            def _():
                swait(rows0, ssem0)

            @pl.when(nb >= 2)
            def _():
                swait(rows1, ssem1)

        def load_chunk(ch, er, ec, ev, sem):
            pltpu.async_copy(row_h.at[pl.ds(base_e + ch * _CE, _CE)], er, sem)
            pltpu.async_copy(col_h.at[pl.ds(base_e + ch * _CE, _CE)], ec, sem)
            pltpu.async_copy(val_h.at[pl.ds(base_e + ch * _CE, _CE)], ev, sem)

        def wait_chunk(er, ec, ev, sem):
            pltpu.make_async_copy(row_h.at[pl.ds(0, _CE)], er, sem).wait()
            pltpu.make_async_copy(col_h.at[pl.ds(0, _CE)], ec, sem).wait()
            pltpu.make_async_copy(val_h.at[pl.ds(0, _CE)], ev, sem).wait()

        for p in range(n_pass):
            lo = p * 2 * r + cid * r

            zbase = sid * zrows
            zds = []
            nfull = zrows // 16
            remz = zrows % 16
            for j in range(nfull):
                zds.append(pltpu.async_copy(
                    zbuf, acc.at[pl.ds(zbase + j * 16, 16)], esemA))
            if remz:
                zds.append(pltpu.async_copy(
                    zbuf.at[pl.ds(0, remz)],
                    acc.at[pl.ds(zbase + nfull * 16, remz)], esemA))
            for dsc in zds:
                dsc.wait()
            plsc.subcore_barrier()

            def process(er, ec, ev, ptr):
                def compact(ei, ptr):
                    off = ei * 16
                    rv = er[pl.ds(off, 16)]
                    lr = rv - lo
                    m = (lr >= 0) & (lr < r)
                    tgt = ptr + plsc.cumsum(ones16, mask=m) - 1
                    plsc.store_scatter(stc, [tgt], ec[pl.ds(off, 16)],
                                       mask=m)
                    plsc.store_scatter(stl, [tgt // _BATCH, tgt % _BATCH], lr,
                                       mask=m)
                    plsc.store_scatter(stv, [tgt], ev[pl.ds(off, 16)],
                                       mask=m)
                    return ptr + jnp.sum(m.astype(jnp.int32))

                ptr = lax.fori_loop(0, _CE // 16, compact, ptr)
                nb = ptr // _BATCH
                run_batches(nb)

                rem_base = nb * _BATCH
                for j in range(_BATCH // 16):
                    vc = stc[pl.ds(rem_base + j * 16, 16)]
                    vl = stl[nb, pl.ds(j * 16, 16)]
                    vv = stv[pl.ds(rem_base + j * 16, 16)]
                    stc[pl.ds(j * 16, 16)] = vc
                    stl[0, pl.ds(j * 16, 16)] = vl
                    stv[pl.ds(j * 16, 16)] = vv
                return ptr - nb * _BATCH

            load_chunk(0, erA, ecA, evA, esemA)

            def cpair(k, ptr):
                c0 = 2 * k
                load_chunk(c0 + 1, erB, ecB, evB, esemB)
                wait_chunk(erA, ecA, evA, esemA)
                ptr = process(erA, ecA, evA, ptr)

                @pl.when(c0 + 2 < nchunk)
                def _():
                    load_chunk(c0 + 2, erA, ecA, evA, esemA)

                wait_chunk(erB, ecB, evB, esemB)
                ptr = process(erB, ecB, evB, ptr)
                return ptr

            ptr = lax.fori_loop(0, nchunk // 2, cpair, jnp.int32(0))

            for j in range(_BATCH // 16):
                f = ptr + j * 16 + iota
                plsc.store_scatter(stc, [f], iota + j * 16)
                plsc.store_scatter(stl, [f // _BATCH, f % _BATCH], iota + r)
                plsc.store_scatter(stv, [f], jnp.zeros((16,), jnp.float32))

            @pl.when(ptr > 0)
            def _():
                gissue(0, rows0, gsem0)
                gwait(0, rows0, gsem0)
                scale(0, rows0)
                pltpu.sync_copy(rows0, acc.at[stl.at[0]], add=True)

            plsc.subcore_barrier()

            start = lo + sid * rpt
            local = sid * rpt

            @pl.when(start + rpt <= n_dst)
            def _():
                pltpu.sync_copy(acc.at[pl.ds(local, rpt)],
                                out_h.at[pl.ds(start, rpt)])

            @pl.when((start < n_dst) & (start + rpt > n_dst))
            def _():
                def chunk(j, _):
                    @pl.when(start + j * 16 + 16 <= n_dst)
                    def _():
                        pltpu.sync_copy(
                            acc.at[pl.ds(local + j * 16, 16)],
                            out_h.at[pl.ds(start + j * 16, 16)])
                    return 0

                lax.fori_loop(0, rpt // 16, chunk, 0)

            plsc.subcore_barrier()

    return spmm_kernel


def _pad_edges(dst, src, val, e_pad):
    e = dst.shape[0]
    dst = dst.astype(jnp.int32)
    src = src.astype(jnp.int32)
    if e_pad != e:
        dst = jnp.pad(dst, (0, e_pad - e), constant_values=0x3FFFFFFF)
        src = jnp.pad(src, (0, e_pad - e))
        val = jnp.pad(val, (0, e_pad - e))
    return dst, src, val


def sc_spmm(dst, src, val, x, n_dst):
    e_pad = _round_up(dst.shape[0], _NS * _CE * 2)
    dst, src, val = _pad_edges(dst, src, val, e_pad)
    return _sc_spmm(e_pad, x.shape[0], n_dst)(dst, src, val, x)


def _combine_body(nin, act, *refs):
    out_ref = refs[-1]
    acc = jnp.zeros_like(out_ref)
    for i in range(nin):
        a = refs[2 * i][...]
        w = refs[2 * i + 1][...]
        acc = acc + jnp.dot(a, w, preferred_element_type=jnp.float32)
    if len(refs) == 2 * nin + 2:
        acc = acc + refs[2 * nin][...]
    if act:
        acc = jax.nn.sigmoid(acc)
    out_ref[...] = acc


def combine(aggs, ws, extra=None, act=True, block=2000):
    n = aggs[0].shape[0]
    nin = len(aggs)
    grid = (pl.cdiv(n, block),)
    in_specs = []
    args = []
    for a, w in zip(aggs, ws):
        in_specs.append(pl.BlockSpec((block, D), lambda i: (i, 0)))
        in_specs.append(pl.BlockSpec((D, D), lambda i: (0, 0)))
        args.extend([a, w])
    if extra is not None:
        in_specs.append(pl.BlockSpec((block, D), lambda i: (i, 0)))
        args.append(extra)
    return pl.pallas_call(
        functools.partial(_combine_body, nin, act),
        grid=grid,
        in_specs=in_specs,
        out_specs=pl.BlockSpec((block, D), lambda i: (i, 0)),
        out_shape=jax.ShapeDtypeStruct((n, D), jnp.float32),
    )(*args)


def _seg_stats_body(x_ref, b_ref, sum_ref, cnt_ref):
    i = pl.program_id(0)
    onehot = (b_ref[...] == lax.broadcasted_iota(jnp.int32, (1, G), 1)).astype(
        jnp.float32
    )
    part = lax.dot_general(onehot, x_ref[...], (((0,), (0,)), ((), ())),
                           preferred_element_type=jnp.float32)
    cpart = jnp.sum(onehot, axis=0, keepdims=True)

    @pl.when(i == 0)
    def _():
        sum_ref[...] = jnp.zeros_like(sum_ref)
        cnt_ref[...] = jnp.zeros_like(cnt_ref)

    sum_ref[...] += part
    cnt_ref[...] += cpart


def _center_body(x_ref, b_ref, mean_ref, ms_ref, out_ref, sq_ref):
    i = pl.program_id(0)
    onehot = (b_ref[...] == lax.broadcasted_iota(jnp.int32, (1, G), 1)).astype(
        jnp.float32
    )
    mean_rows = jnp.dot(onehot, mean_ref[...], preferred_element_type=jnp.float32)
    out = x_ref[...] - mean_rows * ms_ref[...]
    out_ref[...] = out
    part = lax.dot_general(onehot, out * out, (((0,), (0,)), ((), ())),
                           preferred_element_type=jnp.float32)

    @pl.when(i == 0)
    def _():
        sq_ref[...] = jnp.zeros_like(sq_ref)

    sq_ref[...] += part


def _norm_body(o_ref, b_ref, istd_ref, w_ref, bias_ref, out_ref):
    onehot = (b_ref[...] == lax.broadcasted_iota(jnp.int32, (1, G), 1)).astype(
        jnp.float32
    )
    istd_rows = jnp.dot(onehot, istd_ref[...], preferred_element_type=jnp.float32)
    out_ref[...] = w_ref[...] * o_ref[...] * istd_rows + bias_ref[...]


def graph_norm(x, batch2d, weight, bias, mean_scale, block=2000, eps=1e-5):
    n = x.shape[0]
    grid = (pl.cdiv(n, block),)
    row_spec = pl.BlockSpec((block, D), lambda i: (i, 0))
    b_spec = pl.BlockSpec((block, 1), lambda i: (i, 0))
    g_spec = pl.BlockSpec((G, D), lambda i: (0, 0))
    c_spec = pl.BlockSpec((1, G), lambda i: (0, 0))
    d_spec = pl.BlockSpec((1, D), lambda i: (0, 0))

    sums, cnt = pl.pallas_call(
        _seg_stats_body,
        grid=grid,
        in_specs=[row_spec, b_spec],
        out_specs=[g_spec, c_spec],
        out_shape=[
            jax.ShapeDtypeStruct((G, D), jnp.float32),
            jax.ShapeDtypeStruct((1, G), jnp.float32),
        ],
    )(x, batch2d)
    cnt = jnp.maximum(cnt, 1.0)
    mean = sums / cnt.reshape(G, 1)

    out, sq = pl.pallas_call(
        _center_body,
        grid=grid,
        in_specs=[row_spec, b_spec, g_spec, d_spec],
        out_specs=[row_spec, g_spec],
        out_shape=[
            jax.ShapeDtypeStruct((n, D), jnp.float32),
            jax.ShapeDtypeStruct((G, D), jnp.float32),
        ],
    )(x, batch2d, mean, mean_scale.reshape(1, D))
    var = sq / cnt.reshape(G, 1)
    istd = lax.rsqrt(var + eps)

    return pl.pallas_call(
        _norm_body,
        grid=grid,
        in_specs=[row_spec, b_spec, g_spec, d_spec, d_spec],
        out_specs=row_spec,
        out_shape=jax.ShapeDtypeStruct((n, D), jnp.float32),
    )(out, batch2d, istd, weight.reshape(1, D), bias.reshape(1, D))


def spmm(row, col, val, x, n_rows):
    return sc_spmm(row, col, val, x, n_rows)


def kernel(x_0, x_1, x_2, inc1_row, inc1_col, inc1_val, inc2_row, inc2_col,
           inc2_val, l0_row, l0_col, l0_val, l1_row, l1_col, l1_val, l2_row,
           l2_col, l2_val, batch_0, batch_1, params):
    x0, x1, x2 = x_0, x_1, x_2
    for l in range(2):
        p = params['layer_%d' % l]
        a_l0 = spmm(l0_row, l0_col, l0_val, x0, N0)
        a_i1 = spmm(inc1_row, inc1_col, inc1_val, x1, N0)
        a_l1 = spmm(l1_row, l1_col, l1_val, x1, N1)
        a_i1t = spmm(inc1_col, inc1_row, inc1_val, x0, N1)
        a_i2 = spmm(inc2_row, inc2_col, inc2_val, x2, N1)
        a_l2 = spmm(l2_row, l2_col, l2_val, x2, N2)
        a_i2t = spmm(inc2_col, inc2_row, inc2_val, x1, N2)
        x0 = combine([a_l0, a_i1], [p['W_same_0'], p['W_h2l_0']])
        x1 = combine([a_l1, a_i1t, a_i2],
                     [p['W_same_1'], p['W_l2h_1'], p['W_h2l_1']])
        x2 = combine([a_l2, a_i2t], [p['W_same_2'], p['W_l2h_2']])

    b1 = batch_1.astype(jnp.int32).reshape(N1, 1)
    b0 = batch_0.astype(jnp.int32).reshape(N0, 1)

    a1 = spmm(inc2_row, inc2_col, inc2_val, x2, N1)
    pre1 = combine([a1], [params['W_agg_1']], extra=x1, act=False)
    x1_out = graph_norm(pre1, b1, params['gn1_w'], params['gn1_b'],
                        params['gn1_ms'])
    a2 = spmm(inc1_row, inc1_col, inc1_val, x1_out, N0)
    pre0 = combine([a2], [params['W_agg_2']], extra=x0, act=False)
    x0_out = graph_norm(pre0, b0, params['gn2_w'], params['gn2_b'],
                        params['gn2_ms'])
    return (x0_out, x1_out, x2)


# speedup vs baseline: 1.0537x; 1.0537x over previous
import functools
import math
import jax
import jax.numpy as jnp
from jax import lax
from jax.experimental import pallas as pl
from jax.experimental.pallas import tpu as pltpu
from jax.experimental.pallas import tpu_sc as plsc

N0, N1, N2, D, G = 10000, 80000, 20000, 128, 64


_NC, _NS = 2, 16
_NW = _NC * _NS
_BATCH = 128
_CE = 1024
_NST = _CE + 2 * _BATCH
_R_CAP = 11520


def _round_up(x, m):
    return (x + m - 1) // m * m


@functools.lru_cache(maxsize=None)
def _sc_spmm(e_pad, n_src, n_dst):
    assert e_pad % (_NS * _CE * 2) == 0
    ew = e_pad // _NS
    nchunk = ew // _CE
    n_pass = math.ceil(n_dst / (2 * _R_CAP))
    r = _round_up(math.ceil(n_dst / (2 * n_pass)), 256)
    rpt = r // 16
    zrows = rpt + 1

    mesh = plsc.VectorSubcoreMesh(core_axis_name="c", subcore_axis_name="s")

    @functools.partial(
        pl.kernel,
        out_type=jax.ShapeDtypeStruct((n_dst, D), jnp.float32),
        mesh=mesh,
        scratch_types=[
            pltpu.VMEM_SHARED((r + 16, D), jnp.float32),
            pltpu.VMEM((_CE,), jnp.int32),
            pltpu.VMEM((_CE,), jnp.int32),
            pltpu.VMEM((_CE,), jnp.float32),
            pltpu.VMEM((_CE,), jnp.int32),
            pltpu.VMEM((_CE,), jnp.int32),
            pltpu.VMEM((_CE,), jnp.float32),
            pltpu.VMEM((_NST,), jnp.int32),
            pltpu.VMEM((_NST // _BATCH + 1, _BATCH), jnp.int32),
            pltpu.VMEM((_NST,), jnp.float32),
            pltpu.VMEM((_BATCH, D), jnp.float32),
            pltpu.VMEM((_BATCH, D), jnp.float32),
            pltpu.VMEM((16, D), jnp.float32),
            pltpu.SemaphoreType.DMA,
            pltpu.SemaphoreType.DMA,
            pltpu.SemaphoreType.DMA,
            pltpu.SemaphoreType.DMA,
            pltpu.SemaphoreType.DMA,
            pltpu.SemaphoreType.DMA,
        ],
        compiler_params=pltpu.CompilerParams(needs_layout_passes=False),
    )
    def spmm_kernel(row_h, col_h, val_h, x_h, out_h, acc,
                    erA, ecA, evA, erB, ecB, evB,
                    stc, stl, stv, rows0, rows1, zbuf,
                    gsem0, gsem1, ssem0, ssem1, esemA, esemB):
        cid = lax.axis_index("c")
        sid = lax.axis_index("s")
        base_e = sid * ew
        iota = lax.broadcasted_iota(jnp.int32, (16,), 0)
        ones16 = jnp.ones((16,), jnp.int32)
        splat_idx = [jnp.full((16, 1), kk, jnp.int32) for kk in range(16)]
        dnums = lax.GatherDimensionNumbers(
            offset_dims=(), collapsed_slice_dims=(0,), start_index_map=(0,))

        def zero_z(i, _):
            for d in range(D // 16):
                zbuf[i, pl.ds(d * 16, 16)] = jnp.zeros((16,), jnp.float32)
            return 0

        lax.fori_loop(0, 16, zero_z, 0)

        def gissue(bi, rbuf, sem):
            pltpu.async_copy(x_h.at[stc.at[pl.ds(bi * _BATCH, _BATCH)]],
                             rbuf, sem)

        def gwait(bi, rbuf, sem):
            pltpu.make_async_copy(x_h.at[stc.at[pl.ds(bi * _BATCH, _BATCH)]],
                                  rbuf, sem).wait()

        def scale(bi, rbuf):
            boff = bi * _BATCH

            def body(k, _):
                vv = stv[pl.ds(boff + k * 16, 16)]
                for kk in range(16):
                    v = lax.gather(
                        vv, splat_idx[kk], dnums, (1,),
                        mode=lax.GatherScatterMode.PROMISE_IN_BOUNDS)
                    j = k * 16 + kk
                    for d in range(D // 16):
                        rbuf[j, pl.ds(d * 16, 16)] = (
                            rbuf[j, pl.ds(d * 16, 16)] * v)
                return 0

            lax.fori_loop(0, _BATCH // 16, body, 0)

        def swait(rbuf, sem):
            pltpu.make_async_copy(rbuf, acc.at[stl.at[0]], sem).wait()

        def run_batches(nb):
            def pair(k, _):
                b0 = 2 * k

                @pl.when(b0 < nb)
                def _():
                    gwait(b0, rows0, gsem0)
                    scale(b0, rows0)
                    pltpu.async_copy(rows0, acc.at[stl.at[b0]], ssem0,
                                     add=True)

                @pl.when(b0 + 2 < nb)
                def _():
                    swait(rows0, ssem0)
                    gissue(b0 + 2, rows0, gsem0)

                @pl.when(b0 + 1 < nb)
                def _():
                    gwait(b0 + 1, rows1, gsem1)
                    scale(b0 + 1, rows1)
                    pltpu.async_copy(rows1, acc.at[stl.at[b0 + 1]], ssem1,
                                     add=True)

                @pl.when(b0 + 3 < nb)
                def _():
                    swait(rows1, ssem1)
                    gissue(b0 + 3, rows1, gsem1)

                return 0

            lax.fori_loop(0, (nb + 1) // 2, pair, 0)

            @pl.when(nb >= 1)
            def _():
                swait(rows0, ssem0)

            @pl.when(nb >= 2)
            def _():
                swait(rows1, ssem1)

        def load_chunk(ch, er, ec, ev, sem):
            pltpu.async_copy(row_h.at[pl.ds(base_e + ch * _CE, _CE)], er, sem)
            pltpu.async_copy(col_h.at[pl.ds(base_e + ch * _CE, _CE)], ec, sem)
            pltpu.async_copy(val_h.at[pl.ds(base_e + ch * _CE, _CE)], ev, sem)

        def wait_chunk(er, ec, ev, sem):
            pltpu.make_async_copy(row_h.at[pl.ds(0, _CE)], er, sem).wait()
            pltpu.make_async_copy(col_h.at[pl.ds(0, _CE)], ec, sem).wait()
            pltpu.make_async_copy(val_h.at[pl.ds(0, _CE)], ev, sem).wait()

        for p in range(n_pass):
            lo = p * 2 * r + cid * r

            zbase = sid * zrows
            zds = []
            nfull = zrows // 16
            remz = zrows % 16
            for j in range(nfull):
                zds.append(pltpu.async_copy(
                    zbuf, acc.at[pl.ds(zbase + j * 16, 16)], esemA))
            if remz:
                zds.append(pltpu.async_copy(
                    zbuf.at[pl.ds(0, remz)],
                    acc.at[pl.ds(zbase + nfull * 16, remz)], esemA))
            for dsc in zds:
                dsc.wait()
            plsc.subcore_barrier()

            def process(er, ec, ev, ptr):
                def compact(ei, ptr):
                    off = ei * 16
                    rv = er[pl.ds(off, 16)]
                    lr = rv - lo
                    m = (lr >= 0) & (lr < r)
                    tgt = ptr + plsc.cumsum(ones16, mask=m) - 1
                    plsc.store_scatter(stc, [tgt], ec[pl.ds(off, 16)],
                                       mask=m)
                    plsc.store_scatter(stl, [tgt // _BATCH, tgt % _BATCH], lr,
                                       mask=m)
                    plsc.store_scatter(stv, [tgt], ev[pl.ds(off, 16)],
                                       mask=m)
                    new_ptr = ptr + jnp.sum(m.astype(jnp.int32))

                    @pl.when((ptr < _BATCH) & (new_ptr >= _BATCH))
                    def _():
                        gissue(0, rows0, gsem0)

                    @pl.when((ptr < 2 * _BATCH) & (new_ptr >= 2 * _BATCH))
                    def _():
                        gissue(1, rows1, gsem1)

                    return new_ptr

                ptr = lax.fori_loop(0, _CE // 16, compact, ptr)
                nb = ptr // _BATCH
                run_batches(nb)

                rem_base = nb * _BATCH
                for j in range(_BATCH // 16):
                    vc = stc[pl.ds(rem_base + j * 16, 16)]
                    vl = stl[nb, pl.ds(j * 16, 16)]
                    vv = stv[pl.ds(rem_base + j * 16, 16)]
                    stc[pl.ds(j * 16, 16)] = vc
                    stl[0, pl.ds(j * 16, 16)] = vl
                    stv[pl.ds(j * 16, 16)] = vv
                return ptr - nb * _BATCH

            load_chunk(0, erA, ecA, evA, esemA)

            def cpair(k, ptr):
                c0 = 2 * k
                load_chunk(c0 + 1, erB, ecB, evB, esemB)
                wait_chunk(erA, ecA, evA, esemA)
                ptr = process(erA, ecA, evA, ptr)

                @pl.when(c0 + 2 < nchunk)
                def _():
                    load_chunk(c0 + 2, erA, ecA, evA, esemA)

                wait_chunk(erB, ecB, evB, esemB)
                ptr = process(erB, ecB, evB, ptr)
                return ptr

            ptr = lax.fori_loop(0, nchunk // 2, cpair, jnp.int32(0))

            for j in range(_BATCH // 16):
                f = ptr + j * 16 + iota
                plsc.store_scatter(stc, [f], iota + j * 16)
                plsc.store_scatter(stl, [f // _BATCH, f % _BATCH], iota + r)
                plsc.store_scatter(stv, [f], jnp.zeros((16,), jnp.float32))

            @pl.when(ptr > 0)
            def _():
                gissue(0, rows0, gsem0)
                gwait(0, rows0, gsem0)
                scale(0, rows0)
                pltpu.sync_copy(rows0, acc.at[stl.at[0]], add=True)

            plsc.subcore_barrier()

            start = lo + sid * rpt
            local = sid * rpt

            @pl.when(start + rpt <= n_dst)
            def _():
                pltpu.sync_copy(acc.at[pl.ds(local, rpt)],
                                out_h.at[pl.ds(start, rpt)])

            @pl.when((start < n_dst) & (start + rpt > n_dst))
            def _():
                def chunk(j, _):
                    @pl.when(start + j * 16 + 16 <= n_dst)
                    def _():
                        pltpu.sync_copy(
                            acc.at[pl.ds(local + j * 16, 16)],
                            out_h.at[pl.ds(start + j * 16, 16)])
                    return 0

                lax.fori_loop(0, rpt // 16, chunk, 0)

            plsc.subcore_barrier()

    return spmm_kernel


def _pad_edges(dst, src, val, e_pad):
    e = dst.shape[0]
    dst = dst.astype(jnp.int32)
    src = src.astype(jnp.int32)
    if e_pad != e:
        dst = jnp.pad(dst, (0, e_pad - e), constant_values=0x3FFFFFFF)
        src = jnp.pad(src, (0, e_pad - e))
        val = jnp.pad(val, (0, e_pad - e))
    return dst, src, val


def sc_spmm(dst, src, val, x, n_dst):
    e_pad = _round_up(dst.shape[0], _NS * _CE * 2)
    dst, src, val = _pad_edges(dst, src, val, e_pad)
    return _sc_spmm(e_pad, x.shape[0], n_dst)(dst, src, val, x)


def _combine_body(nin, act, *refs):
    out_ref = refs[-1]
    acc = jnp.zeros_like(out_ref)
    for i in range(nin):
        a = refs[2 * i][...]
        w = refs[2 * i + 1][...]
        acc = acc + jnp.dot(a, w, preferred_element_type=jnp.float32)
    if len(refs) == 2 * nin + 2:
        acc = acc + refs[2 * nin][...]
    if act:
        acc = jax.nn.sigmoid(acc)
    out_ref[...] = acc


def combine(aggs, ws, extra=None, act=True, block=2000):
    n = aggs[0].shape[0]
    nin = len(aggs)
    grid = (pl.cdiv(n, block),)
    in_specs = []
    args = []
    for a, w in zip(aggs, ws):
        in_specs.append(pl.BlockSpec((block, D), lambda i: (i, 0)))
        in_specs.append(pl.BlockSpec((D, D), lambda i: (0, 0)))
        args.extend([a, w])
    if extra is not None:
        in_specs.append(pl.BlockSpec((block, D), lambda i: (i, 0)))
        args.append(extra)
    return pl.pallas_call(
        functools.partial(_combine_body, nin, act),
        grid=grid,
        in_specs=in_specs,
        out_specs=pl.BlockSpec((block, D), lambda i: (i, 0)),
        out_shape=jax.ShapeDtypeStruct((n, D), jnp.float32),
    )(*args)


def _seg_stats_body(x_ref, b_ref, sum_ref, cnt_ref):
    i = pl.program_id(0)
    onehot = (b_ref[...] == lax.broadcasted_iota(jnp.int32, (1, G), 1)).astype(
        jnp.float32
    )
    part = lax.dot_general(onehot, x_ref[...], (((0,), (0,)), ((), ())),
                           preferred_element_type=jnp.float32)
    cpart = jnp.sum(onehot, axis=0, keepdims=True)

    @pl.when(i == 0)
    def _():
        sum_ref[...] = jnp.zeros_like(sum_ref)
        cnt_ref[...] = jnp.zeros_like(cnt_ref)

    sum_ref[...] += part
    cnt_ref[...] += cpart


def _center_body(x_ref, b_ref, mean_ref, ms_ref, out_ref, sq_ref):
    i = pl.program_id(0)
    onehot = (b_ref[...] == lax.broadcasted_iota(jnp.int32, (1, G), 1)).astype(
        jnp.float32
    )
    mean_rows = jnp.dot(onehot, mean_ref[...], preferred_element_type=jnp.float32)
    out = x_ref[...] - mean_rows * ms_ref[...]
    out_ref[...] = out
    part = lax.dot_general(onehot, out * out, (((0,), (0,)), ((), ())),
                           preferred_element_type=jnp.float32)

    @pl.when(i == 0)
    def _():
        sq_ref[...] = jnp.zeros_like(sq_ref)

    sq_ref[...] += part


def _norm_body(o_ref, b_ref, istd_ref, w_ref, bias_ref, out_ref):
    onehot = (b_ref[...] == lax.broadcasted_iota(jnp.int32, (1, G), 1)).astype(
        jnp.float32
    )
    istd_rows = jnp.dot(onehot, istd_ref[...], preferred_element_type=jnp.float32)
    out_ref[...] = w_ref[...] * o_ref[...] * istd_rows + bias_ref[...]


def graph_norm(x, batch2d, weight, bias, mean_scale, block=2000, eps=1e-5):
    n = x.shape[0]
    grid = (pl.cdiv(n, block),)
    row_spec = pl.BlockSpec((block, D), lambda i: (i, 0))
    b_spec = pl.BlockSpec((block, 1), lambda i: (i, 0))
    g_spec = pl.BlockSpec((G, D), lambda i: (0, 0))
    c_spec = pl.BlockSpec((1, G), lambda i: (0, 0))
    d_spec = pl.BlockSpec((1, D), lambda i: (0, 0))

    sums, cnt = pl.pallas_call(
        _seg_stats_body,
        grid=grid,
        in_specs=[row_spec, b_spec],
        out_specs=[g_spec, c_spec],
        out_shape=[
            jax.ShapeDtypeStruct((G, D), jnp.float32),
            jax.ShapeDtypeStruct((1, G), jnp.float32),
        ],
    )(x, batch2d)
    cnt = jnp.maximum(cnt, 1.0)
    mean = sums / cnt.reshape(G, 1)

    out, sq = pl.pallas_call(
        _center_body,
        grid=grid,
        in_specs=[row_spec, b_spec, g_spec, d_spec],
        out_specs=[row_spec, g_spec],
        out_shape=[
            jax.ShapeDtypeStruct((n, D), jnp.float32),
            jax.ShapeDtypeStruct((G, D), jnp.float32),
        ],
    )(x, batch2d, mean, mean_scale.reshape(1, D))
    var = sq / cnt.reshape(G, 1)
    istd = lax.rsqrt(var + eps)

    return pl.pallas_call(
        _norm_body,
        grid=grid,
        in_specs=[row_spec, b_spec, g_spec, d_spec, d_spec],
        out_specs=row_spec,
        out_shape=jax.ShapeDtypeStruct((n, D), jnp.float32),
    )(out, batch2d, istd, weight.reshape(1, D), bias.reshape(1, D))


def spmm(row, col, val, x, n_rows):
    return sc_spmm(row, col, val, x, n_rows)


def kernel(x_0, x_1, x_2, inc1_row, inc1_col, inc1_val, inc2_row, inc2_col,
           inc2_val, l0_row, l0_col, l0_val, l1_row, l1_col, l1_val, l2_row,
           l2_col, l2_val, batch_0, batch_1, params):
    x0, x1, x2 = x_0, x_1, x_2
    for l in range(2):
        p = params['layer_%d' % l]
        a_l0 = spmm(l0_row, l0_col, l0_val, x0, N0)
        a_i1 = spmm(inc1_row, inc1_col, inc1_val, x1, N0)
        a_l1 = spmm(l1_row, l1_col, l1_val, x1, N1)
        a_i1t = spmm(inc1_col, inc1_row, inc1_val, x0, N1)
        a_i2 = spmm(inc2_row, inc2_col, inc2_val, x2, N1)
        a_l2 = spmm(l2_row, l2_col, l2_val, x2, N2)
        a_i2t = spmm(inc2_col, inc2_row, inc2_val, x1, N2)
        x0 = combine([a_l0, a_i1], [p['W_same_0'], p['W_h2l_0']])
        x1 = combine([a_l1, a_i1t, a_i2],
                     [p['W_same_1'], p['W_l2h_1'], p['W_h2l_1']])
        x2 = combine([a_l2, a_i2t], [p['W_same_2'], p['W_l2h_2']])

    b1 = batch_1.astype(jnp.int32).reshape(N1, 1)
    b0 = batch_0.astype(jnp.int32).reshape(N0, 1)

    a1 = spmm(inc2_row, inc2_col, inc2_val, x2, N1)
    pre1 = combine([a1], [params['W_agg_1']], extra=x1, act=False)
    x1_out = graph_norm(pre1, b1, params['gn1_w'], params['gn1_b'],
                        params['gn1_ms'])
    a2 = spmm(inc1_row, inc1_col, inc1_val, x1_out, N0)
    pre0 = combine([a2], [params['W_agg_2']], extra=x0, act=False)
    x0_out = graph_norm(pre0, b0, params['gn2_w'], params['gn2_b'],
                        params['gn2_ms'])
    return (x0_out, x1_out, x2)
